# Initial kernel scaffold; baseline (speedup 1.0000x reference)
import jax
import jax.numpy as jnp
from jax.experimental import pallas as pl


def kernel(hidden_states, gate_w, w_gate, w_up, w_down, sw_gate, sw_up, sw_down):
    raise NotImplementedError("write your pallas kernel here")


# speedup vs baseline: 1.5070x; 1.5070x over previous
import functools
import jax
import jax.numpy as jnp
from jax.experimental import pallas as pl
from jax.experimental.pallas import tpu as pltpu

HIDDEN = 1024
E = 8
TOP_K = 2
D_FF = 512
N_GROUP = 4
TOPK_GROUP = 2
ROUTED_SCALE = 2.5
T = 2048
SHARED_FF = 1024

TB = 256


def _silu(v):
    return v * (1.0 / (1.0 + jnp.exp(-v)))


def _topk_mask_cols(cols, k, valid=None):
    n = len(cols)
    sels = []
    for i in range(n):
        beaten = None
        for j in range(n):
            if i == j:
                continue
            b = (cols[j] > cols[i]) | ((cols[j] == cols[i]) & (j < i))
            if valid is not None:
                b = b & valid[j]
            bi = b.astype(jnp.int32)
            beaten = bi if beaten is None else beaten + bi
        sel = beaten < k
        if valid is not None:
            sel = sel & valid[i]
        sels.append(sel)
    return sels


def _dense_body(x_ref, gate_ref, wg_ref, wu_ref, wd_ref, swg_ref, swu_ref,
                swd_ref, out_ref):
    xb = x_ref[:]
    logits = jnp.dot(xb, gate_ref[:], preferred_element_type=jnp.float32)

    lcols = [logits[:, e:e + 1] for e in range(E)]
    gcols = [jnp.maximum(lcols[2 * j], lcols[2 * j + 1]) for j in range(N_GROUP)]
    gsel = _topk_mask_cols(gcols, TOPK_GROUP)
    valid = [gsel[e // 2] for e in range(E)]
    esel = _topk_mask_cols(lcols, TOP_K, valid=valid)

    neg = jnp.float32(-1e30)
    mvals = [jnp.where(esel[e], lcols[e], neg) for e in range(E)]
    M = functools.reduce(jnp.maximum, mvals)
    wcols = [jnp.where(esel[e], jnp.exp(lcols[e] - M), 0.0) for e in range(E)]
    wsum = functools.reduce(jnp.add, wcols)
    scale = ROUTED_SCALE / wsum
    dw = [wcols[e] * scale for e in range(E)]

    xbf = xb.astype(jnp.bfloat16)
    acc = jnp.zeros((TB, HIDDEN), jnp.float32)
    for e in range(E):
        g = jnp.dot(xbf, wg_ref[e], preferred_element_type=jnp.float32)
        u = jnp.dot(xbf, wu_ref[e], preferred_element_type=jnp.float32)
        h = (_silu(g) * u).astype(jnp.bfloat16)
        y = jnp.dot(h, wd_ref[e], preferred_element_type=jnp.float32)
        acc = acc + dw[e] * y

    sg = jnp.dot(xbf, swg_ref[:], preferred_element_type=jnp.float32)
    su = jnp.dot(xbf, swu_ref[:], preferred_element_type=jnp.float32)
    hs = (_silu(sg) * su).astype(jnp.bfloat16)
    sh = jnp.dot(hs, swd_ref[:], preferred_element_type=jnp.float32)
    out_ref[:] = acc + sh


def kernel(hidden_states, gate_w, w_gate, w_up, w_down, sw_gate, sw_up,
           sw_down):
    wg = w_gate.astype(jnp.bfloat16)
    wu = w_up.astype(jnp.bfloat16)
    wd = w_down.astype(jnp.bfloat16)
    swg = sw_gate.astype(jnp.bfloat16)
    swu = sw_up.astype(jnp.bfloat16)
    swd = sw_down.astype(jnp.bfloat16)

    full = lambda shape: pl.BlockSpec(shape, lambda i: (0,) * len(shape))
    out = pl.pallas_call(
        _dense_body,
        grid=(T // TB,),
        in_specs=[
            pl.BlockSpec((TB, HIDDEN), lambda i: (i, 0)),
            full((HIDDEN, E)),
            full((E, HIDDEN, D_FF)),
            full((E, HIDDEN, D_FF)),
            full((E, D_FF, HIDDEN)),
            full((HIDDEN, SHARED_FF)),
            full((HIDDEN, SHARED_FF)),
            full((SHARED_FF, HIDDEN)),
        ],
        out_specs=pl.BlockSpec((TB, HIDDEN), lambda i: (i, 0)),
        out_shape=jax.ShapeDtypeStruct((T, HIDDEN), jnp.float32),
        compiler_params=pltpu.CompilerParams(
            dimension_semantics=("arbitrary",),
        ),
    )(hidden_states, gate_w, wg, wu, wd, swg, swu, swd)
    return out
